# Initial kernel scaffold; baseline (speedup 1.0000x reference)
import jax
import jax.numpy as jnp
from jax.experimental import pallas as pl


def kernel(x, w1, b1, w2, b2, ew1, eb1, ew2, eb2, phi):
    raise NotImplementedError("write your pallas kernel here")


# speedup vs baseline: 1.0595x; 1.0595x over previous
import functools

import jax
import jax.numpy as jnp
from jax.experimental import pallas as pl

NPAD = 128
BETA = 1.0


def _slots_body(x_ref, phi_ref, logits_ref, slots_ref):
    xb = x_ref[0]
    lg = jnp.dot(xb, phi_ref[...],
                 preferred_element_type=jnp.float32)
    logits_ref[0] = lg
    m = jnp.max(lg, axis=0, keepdims=True)
    e = jnp.exp(lg - m)
    disp = (e / jnp.sum(e, axis=0, keepdims=True)).astype(xb.dtype)
    slots = jax.lax.dot_general(disp, xb, (((0,), (0,)), ((), ())),
                                preferred_element_type=jnp.float32)
    slots_ref[0] = slots


def _expert_body(se_ref, ew1_ref, eb1_ref, ew2_ref, eb2_ref, out_ref):
    f = pl.program_id(1)

    @pl.when(f == 0)
    def _init():
        out_ref[...] = jnp.broadcast_to(eb2_ref[...], out_ref.shape)

    s = se_ref[0]
    h = jnp.dot(s, ew1_ref[0].astype(s.dtype),
                preferred_element_type=jnp.float32) + eb1_ref[0]
    h = jax.nn.gelu(h).astype(s.dtype)
    acc = jnp.dot(h, ew2_ref[0].astype(s.dtype),
                  preferred_element_type=jnp.float32)
    out_ref[...] += acc[None]


def _fused_body(x_ref, w1_ref, b1_ref, w2_ref, b2_ref, lg_ref, outs_ref,
                y_ref, *, n_real):
    j = pl.program_id(1)

    @pl.when(j == 0)
    def _combine():
        lg = lg_ref[...]
        lane = jax.lax.broadcasted_iota(jnp.int32, lg.shape, 1)
        lg = jnp.where(lane < n_real, lg, -jnp.inf)
        m = jnp.max(lg, axis=1, keepdims=True)
        e = jnp.exp(lg - m)
        comb = (e / jnp.sum(e, axis=1, keepdims=True)).astype(x_ref.dtype)
        moe = jnp.dot(comb, outs_ref[0],
                      preferred_element_type=jnp.float32)
        y_ref[...] = BETA * moe + b2_ref[...]

    x = x_ref[...]
    h = jnp.dot(x, w1_ref[...],
                preferred_element_type=jnp.float32) + b1_ref[...]
    h = jax.nn.gelu(h).astype(x.dtype)
    y_ref[...] += jnp.dot(h, w2_ref[...],
                          preferred_element_type=jnp.float32)


def kernel(x, w1, b1, w2, b2, ew1, eb1, ew2, eb2, phi):
    B, S, D = x.shape
    D_FF = w1.shape[1]
    E = ew1.shape[0]
    N = phi.shape[1]
    NS = N // E
    TOKENS = B * S

    xb16 = x.astype(jnp.bfloat16)
    phi_p = jnp.pad(phi, ((0, 0), (0, NPAD - N))).astype(jnp.bfloat16)

    logits, slots = pl.pallas_call(
        _slots_body,
        grid=(B,),
        in_specs=[
            pl.BlockSpec((1, S, D), lambda b: (b, 0, 0)),
            pl.BlockSpec((D, NPAD), lambda b: (0, 0)),
        ],
        out_specs=[
            pl.BlockSpec((1, S, NPAD), lambda b: (b, 0, 0)),
            pl.BlockSpec((1, NPAD, D), lambda b: (b, 0, 0)),
        ],
        out_shape=[
            jax.ShapeDtypeStruct((B, S, NPAD), jnp.float32),
            jax.ShapeDtypeStruct((B, NPAD, D), jnp.float32),
        ],
    )(xb16, phi_p)

    R = B * NS
    se = (slots[:, :N, :].reshape(B, E, NS, D).transpose(1, 0, 2, 3)
          .reshape(E, R, D).astype(jnp.bfloat16))

    F2 = min(1024, D_FF)
    outs = pl.pallas_call(
        _expert_body,
        grid=(E, D_FF // F2),
        in_specs=[
            pl.BlockSpec((1, R, D), lambda e, f: (e, 0, 0)),
            pl.BlockSpec((1, D, F2), lambda e, f: (e, 0, f)),
            pl.BlockSpec((1, 1, F2), lambda e, f: (e, 0, f)),
            pl.BlockSpec((1, F2, D), lambda e, f: (e, f, 0)),
            pl.BlockSpec((1, 1, D), lambda e, f: (e, 0, 0)),
        ],
        out_specs=pl.BlockSpec((1, R, D), lambda e, f: (e, 0, 0)),
        out_shape=jax.ShapeDtypeStruct((E, R, D), jnp.float32),
    )(se, ew1.reshape(E, D, D_FF), eb1.reshape(E, 1, D_FF),
      ew2, eb2.reshape(E, 1, D))

    outs_b = (outs.reshape(E, B, NS, D).transpose(1, 0, 2, 3)
              .reshape(B, N, D))
    outs_p = jnp.pad(outs_b, ((0, 0), (0, NPAD - N), (0, 0))).astype(
        jnp.bfloat16)

    T = min(1024, S)
    F3 = min(1024, D_FF)
    tpb = S // T
    xf = xb16.reshape(TOKENS, D)
    lgf = logits.reshape(TOKENS, NPAD)
    w1b = w1.astype(jnp.bfloat16)
    w2b = w2.astype(jnp.bfloat16)
    b1r = b1.reshape(1, D_FF)
    b2r = b2.reshape(1, D)

    y = pl.pallas_call(
        functools.partial(_fused_body, n_real=N),
        grid=(TOKENS // T, D_FF // F3),
        in_specs=[
            pl.BlockSpec((T, D), lambda i, j: (i, 0)),
            pl.BlockSpec((D, F3), lambda i, j: (0, j)),
            pl.BlockSpec((1, F3), lambda i, j: (0, j)),
            pl.BlockSpec((F3, D), lambda i, j: (j, 0)),
            pl.BlockSpec((1, D), lambda i, j: (0, 0)),
            pl.BlockSpec((T, NPAD), lambda i, j: (i, 0)),
            pl.BlockSpec((1, NPAD, D), lambda i, j: (i // tpb, 0, 0)),
        ],
        out_specs=pl.BlockSpec((T, D), lambda i, j: (i, 0)),
        out_shape=jax.ShapeDtypeStruct((TOKENS, D), jnp.float32),
    )(xf, w1b, b1r, w2b, b2r, lgf, outs_p)

    return y.reshape(B, S, D)


# speedup vs baseline: 1.1213x; 1.0583x over previous
import functools

import jax
import jax.numpy as jnp
from jax.experimental import pallas as pl

NPAD = 128
BETA = 1.0


def _slots_body(x_ref, phi_ref, logits_ref, slots_ref):
    xb = x_ref[0].astype(jnp.bfloat16)
    lg = jnp.dot(xb, phi_ref[...].astype(jnp.bfloat16),
                 preferred_element_type=jnp.float32)
    logits_ref[0] = lg
    m = jnp.max(lg, axis=0, keepdims=True)
    e = jnp.exp(lg - m)
    disp = (e / jnp.sum(e, axis=0, keepdims=True)).astype(jnp.bfloat16)
    slots = jax.lax.dot_general(disp, xb, (((0,), (0,)), ((), ())),
                                preferred_element_type=jnp.float32)
    slots_ref[0] = slots


def _expert_body(se_ref, ew1_ref, eb1_ref, ew2_ref, eb2_ref, out_ref):
    f = pl.program_id(1)

    @pl.when(f == 0)
    def _init():
        out_ref[...] = jnp.broadcast_to(eb2_ref[...], out_ref.shape)

    s = se_ref[0].astype(jnp.bfloat16)
    h = jnp.dot(s, ew1_ref[0].astype(jnp.bfloat16),
                preferred_element_type=jnp.float32) + eb1_ref[0]
    h = jax.nn.gelu(h).astype(jnp.bfloat16)
    acc = jnp.dot(h, ew2_ref[0].astype(jnp.bfloat16),
                  preferred_element_type=jnp.float32)
    out_ref[...] += acc[None]


def _fused_body(x_ref, w1_ref, b1_ref, w2_ref, b2_ref, lg_ref, outs_ref,
                y_ref, *, n_real):
    j = pl.program_id(1)

    @pl.when(j == 0)
    def _combine():
        lg = lg_ref[...]
        lane = jax.lax.broadcasted_iota(jnp.int32, lg.shape, 1)
        lg = jnp.where(lane < n_real, lg, -jnp.inf)
        m = jnp.max(lg, axis=1, keepdims=True)
        e = jnp.exp(lg - m)
        comb = (e / jnp.sum(e, axis=1, keepdims=True)).astype(jnp.bfloat16)
        moe = jnp.dot(comb, outs_ref[0].astype(jnp.bfloat16),
                      preferred_element_type=jnp.float32)
        y_ref[...] = BETA * moe + b2_ref[...]

    x = x_ref[...].astype(jnp.bfloat16)
    h = jnp.dot(x, w1_ref[...].astype(jnp.bfloat16),
                preferred_element_type=jnp.float32) + b1_ref[...]
    h = jax.nn.gelu(h).astype(jnp.bfloat16)
    y_ref[...] += jnp.dot(h, w2_ref[...].astype(jnp.bfloat16),
                          preferred_element_type=jnp.float32)


def kernel(x, w1, b1, w2, b2, ew1, eb1, ew2, eb2, phi):
    B, S, D = x.shape
    D_FF = w1.shape[1]
    E = ew1.shape[0]
    N = phi.shape[1]
    NS = N // E
    TOKENS = B * S

    phi_p = jnp.pad(phi, ((0, 0), (0, NPAD - N)))

    logits, slots = pl.pallas_call(
        _slots_body,
        grid=(B,),
        in_specs=[
            pl.BlockSpec((1, S, D), lambda b: (b, 0, 0)),
            pl.BlockSpec((D, NPAD), lambda b: (0, 0)),
        ],
        out_specs=[
            pl.BlockSpec((1, S, NPAD), lambda b: (b, 0, 0)),
            pl.BlockSpec((1, NPAD, D), lambda b: (b, 0, 0)),
        ],
        out_shape=[
            jax.ShapeDtypeStruct((B, S, NPAD), jnp.float32),
            jax.ShapeDtypeStruct((B, NPAD, D), jnp.float32),
        ],
    )(x, phi_p)

    R = B * NS
    se = (slots[:, :N, :].reshape(B, E, NS, D).transpose(1, 0, 2, 3)
          .reshape(E, R, D))

    F2 = min(1024, D_FF)
    outs = pl.pallas_call(
        _expert_body,
        grid=(E, D_FF // F2),
        in_specs=[
            pl.BlockSpec((1, R, D), lambda e, f: (e, 0, 0)),
            pl.BlockSpec((1, D, F2), lambda e, f: (e, 0, f)),
            pl.BlockSpec((1, 1, F2), lambda e, f: (e, 0, f)),
            pl.BlockSpec((1, F2, D), lambda e, f: (e, f, 0)),
            pl.BlockSpec((1, 1, D), lambda e, f: (e, 0, 0)),
        ],
        out_specs=pl.BlockSpec((1, R, D), lambda e, f: (e, 0, 0)),
        out_shape=jax.ShapeDtypeStruct((E, R, D), jnp.float32),
    )(se, ew1.reshape(E, D, D_FF), eb1.reshape(E, 1, D_FF),
      ew2, eb2.reshape(E, 1, D))

    outs_b = (outs.reshape(E, B, NS, D).transpose(1, 0, 2, 3)
              .reshape(B, N, D))
    outs_p = jnp.pad(outs_b, ((0, 0), (0, NPAD - N), (0, 0)))

    T = min(1024, S)
    F3 = min(512, D_FF)
    tpb = S // T
    xf = x.reshape(TOKENS, D)
    lgf = logits.reshape(TOKENS, NPAD)
    b1r = b1.reshape(1, D_FF)
    b2r = b2.reshape(1, D)

    y = pl.pallas_call(
        functools.partial(_fused_body, n_real=N),
        grid=(TOKENS // T, D_FF // F3),
        in_specs=[
            pl.BlockSpec((T, D), lambda i, j: (i, 0)),
            pl.BlockSpec((D, F3), lambda i, j: (0, j)),
            pl.BlockSpec((1, F3), lambda i, j: (0, j)),
            pl.BlockSpec((F3, D), lambda i, j: (j, 0)),
            pl.BlockSpec((1, D), lambda i, j: (0, 0)),
            pl.BlockSpec((T, NPAD), lambda i, j: (i, 0)),
            pl.BlockSpec((1, NPAD, D), lambda i, j: (i // tpb, 0, 0)),
        ],
        out_specs=pl.BlockSpec((T, D), lambda i, j: (i, 0)),
        out_shape=jax.ShapeDtypeStruct((TOKENS, D), jnp.float32),
    )(xf, w1, b1r, w2, b2r, lgf, outs_p)

    return y.reshape(B, S, D)
